# baseline (speedup 1.0000x reference)
import functools

import jax
import jax.numpy as jnp
from jax import lax
from jax.experimental import pallas as pl
from jax.experimental.pallas import tpu as pltpu
from jax.experimental.pallas import tpu_sc as plsc

D = 512

_mesh = plsc.VectorSubcoreMesh(
    core_axis_name="c", subcore_axis_name="s", num_cores=1, num_subcores=1
)


@functools.partial(
    pl.kernel,
    mesh=_mesh,
    out_type=(
        jax.ShapeDtypeStruct((1, 3), jnp.float32),
        jax.ShapeDtypeStruct((1, 1), jnp.float32),
        jax.ShapeDtypeStruct((1, 2), jnp.int32),
    ),
    scratch_types=[
        pltpu.VMEM((4,), jnp.int32),
        pltpu.VMEM((16,), jnp.int32),
        pltpu.VMEM((16,), jnp.float32),
        pltpu.VMEM((16,), jnp.float32),
        pltpu.VMEM((16,), jnp.int32),
        pltpu.SemaphoreType.DMA,
        pltpu.SemaphoreType.DMA,
        [pltpu.SemaphoreType.DMA] * 3,
    ],
    compiler_params=pltpu.CompilerParams(needs_layout_passes=False),
)
def _scatter_aux_sc(ev_hbm, mat_ref, nodes_hbm, feat_hbm, edges_hbm,
                    ev_v, chunk_v, aux_v, feat_v, zed_v,
                    sem_ev, sem_chunk, sems_out):
    pltpu.async_copy(ev_hbm, ev_v, sem_ev).wait()
    lane = lax.iota(jnp.int32, 16)
    zero = jnp.zeros((16,), jnp.int32)
    x_vec = plsc.load_gather(ev_v, [zero])
    y_vec = plsc.load_gather(ev_v, [zero + 1])
    x_s = jnp.sum(jnp.where(lane == 0, x_vec, zero), dtype=jnp.int32)
    y_s = jnp.sum(jnp.where(lane == 1, y_vec, zero), dtype=jnp.int32)
    c0 = (y_s // 16) * 16
    pltpu.async_copy(mat_ref.at[x_s, pl.ds(c0, 16)], chunk_v, sem_chunk).wait()
    chunk_v[...] = jnp.where(lane == y_s - c0, zero, chunk_v[...])
    pltpu.sync_copy(chunk_v, mat_ref.at[x_s, pl.ds(c0, 16)])

    aux_v[...] = plsc.load_gather(ev_v, [lane & 3]).astype(jnp.float32)
    feat_v[...] = plsc.load_gather(ev_v, [zero + 3]).astype(jnp.float32)
    zed_v[...] = zero
    i0 = jnp.int32(0)
    cps = [
        pltpu.async_copy(aux_v.at[pl.ds(0, 3)], nodes_hbm.at[i0], sems_out[0]),
        pltpu.async_copy(feat_v.at[pl.ds(0, 1)], feat_hbm.at[i0], sems_out[1]),
        pltpu.async_copy(zed_v.at[pl.ds(0, 2)], edges_hbm.at[i0], sems_out[2]),
    ]
    for cp in cps:
        cp.wait()


def _tc_body(mat_ref, out_ref):
    out_ref[...] = mat_ref[...]


_tc_copy = pl.pallas_call(
    _tc_body,
    in_specs=[pl.BlockSpec(memory_space=pltpu.VMEM)],
    out_specs=pl.BlockSpec(memory_space=pltpu.VMEM),
    out_shape=jax.ShapeDtypeStruct((D, D), jnp.int32),
)


def kernel(event, neighbour_matrix):
    ev = event.astype(jnp.int32)
    mat = _tc_copy(neighbour_matrix)
    mat_ref = jax.new_ref(mat)
    nodes, features, edges = _scatter_aux_sc(ev, mat_ref)
    new_matrix = mat_ref[...]
    return nodes, features, edges, new_matrix


# speedup vs baseline: 1.0040x; 1.0040x over previous
import functools

import jax
import jax.numpy as jnp
from jax import lax
from jax.experimental import pallas as pl
from jax.experimental.pallas import tpu as pltpu
from jax.experimental.pallas import tpu_sc as plsc

D = 512

_mesh = plsc.VectorSubcoreMesh(
    core_axis_name="c", subcore_axis_name="s", num_cores=1, num_subcores=1
)


@functools.partial(
    pl.kernel,
    mesh=_mesh,
    out_type=(
        jax.ShapeDtypeStruct((1, 3), jnp.float32),
        jax.ShapeDtypeStruct((1, 1), jnp.float32),
        jax.ShapeDtypeStruct((1, 2), jnp.int32),
    ),
    scratch_types=[
        pltpu.VMEM((4,), jnp.int32),
        pltpu.VMEM((16,), jnp.int32),
        pltpu.VMEM((16,), jnp.float32),
        pltpu.VMEM((16,), jnp.float32),
        pltpu.VMEM((16,), jnp.int32),
        pltpu.SemaphoreType.DMA,
        pltpu.SemaphoreType.DMA,
        [pltpu.SemaphoreType.DMA] * 3,
    ],
    compiler_params=pltpu.CompilerParams(needs_layout_passes=False),
)
def _graphgen_sc(ev_hbm, mat_ref, nodes_hbm, feat_hbm, edges_hbm,
                 ev_v, chunk_v, aux_v, feat_v, zed_v,
                 sem_ev, sem_chunk, sems_out):
    pltpu.async_copy(ev_hbm, ev_v, sem_ev).wait()
    lane = lax.iota(jnp.int32, 16)
    zero = jnp.zeros((16,), jnp.int32)
    x_vec = plsc.load_gather(ev_v, [zero])
    y_vec = plsc.load_gather(ev_v, [zero + 1])
    x_s = jnp.sum(jnp.where(lane == 0, x_vec, zero), dtype=jnp.int32)
    y_s = jnp.sum(jnp.where(lane == 1, y_vec, zero), dtype=jnp.int32)
    c0 = (y_s // 16) * 16
    pltpu.async_copy(mat_ref.at[x_s, pl.ds(c0, 16)], chunk_v, sem_chunk).wait()
    chunk_v[...] = jnp.where(lane == y_s - c0, zero, chunk_v[...])
    pltpu.sync_copy(chunk_v, mat_ref.at[x_s, pl.ds(c0, 16)])

    aux_v[...] = plsc.load_gather(ev_v, [lane & 3]).astype(jnp.float32)
    feat_v[...] = plsc.load_gather(ev_v, [zero + 3]).astype(jnp.float32)
    zed_v[...] = zero
    i0 = jnp.int32(0)
    cps = [
        pltpu.async_copy(aux_v.at[pl.ds(0, 3)], nodes_hbm.at[i0], sems_out[0]),
        pltpu.async_copy(feat_v.at[pl.ds(0, 1)], feat_hbm.at[i0], sems_out[1]),
        pltpu.async_copy(zed_v.at[pl.ds(0, 2)], edges_hbm.at[i0], sems_out[2]),
    ]
    for cp in cps:
        cp.wait()


def _tc_copy_body(mat_ref, out_ref):
    out_ref[...] = mat_ref[...]


_tc_copy = pl.pallas_call(
    _tc_copy_body,
    in_specs=[pl.BlockSpec(memory_space=pltpu.VMEM)],
    out_specs=pl.BlockSpec(memory_space=pltpu.VMEM),
    out_shape=jax.ShapeDtypeStruct((D, D), jnp.int32),
)


def kernel(event, neighbour_matrix):
    ev = event.astype(jnp.int32)
    mat = _tc_copy(neighbour_matrix)
    mat_ref = jax.new_ref(mat)
    nodes, features, edges = _graphgen_sc(ev, mat_ref)
    new_matrix = mat_ref[...]
    return nodes, features, edges, new_matrix
